# Initial kernel scaffold; baseline (speedup 1.0000x reference)
import functools

import jax
import jax.numpy as jnp
from jax import lax
from jax.experimental import pallas as pl
from jax.experimental.pallas import tpu as pltpu
from jax.experimental.pallas import tpu_sc as plsc

_N = 10000
_E = 320000
_C = 128
_H = 8
_CH = 16
_ED = 16

_NC = 2
_NS = 16
_NW = _NC * _NS
_EPW = _E // _NW
_CHUNK = 80
_NCHUNK = _EPW // _CHUNK
_ACCW = 144
_RPS = _N // _NS
_ZROWS = 125


def _node_transform(x, Wl, bl, Wr, br):
    def body(x_ref, wl_ref, bl_ref, wr_ref, br_ref, xl_ref, xr_ref):
        xv = x_ref[...]
        xl_ref[...] = (
            jnp.dot(xv, wl_ref[...], preferred_element_type=jnp.float32)
            + bl_ref[...]
        )
        xr_ref[...] = (
            jnp.dot(xv, wr_ref[...], preferred_element_type=jnp.float32)
            + br_ref[...]
        )

    return pl.pallas_call(
        body,
        out_shape=(
            jax.ShapeDtypeStruct((_N, _C), jnp.float32),
            jax.ShapeDtypeStruct((_N, _C), jnp.float32),
        ),
    )(x, Wl, bl.reshape(1, _C), Wr, br.reshape(1, _C))


def _edge_transform(ea, We):
    blk = 8000

    def body(ea_ref, we_ref, xe_ref):
        xe_ref[...] = jnp.dot(
            ea_ref[...], we_ref[...], preferred_element_type=jnp.float32
        )

    return pl.pallas_call(
        body,
        grid=(_E // blk,),
        in_specs=[
            pl.BlockSpec((blk, _ED), lambda i: (i, 0)),
            pl.BlockSpec((_ED, _C), lambda i: (0, 0)),
        ],
        out_specs=pl.BlockSpec((blk, _C), lambda i: (i, 0)),
        out_shape=jax.ShapeDtypeStruct((_E, _C), jnp.float32),
    )(ea, We)


def _sc_edge_pass(xl, xr, xe, src, dst, att):
    mesh = plsc.VectorSubcoreMesh(core_axis_name="c", subcore_axis_name="s")

    @functools.partial(
        pl.kernel,
        out_type=jax.ShapeDtypeStruct((_NC * _N, _ACCW), jnp.float32),
        mesh=mesh,
        scratch_types=[
            pltpu.VMEM((_CHUNK,), jnp.int32),
            pltpu.VMEM((_CHUNK,), jnp.int32),
            pltpu.VMEM((_CHUNK, _C), jnp.float32),
            pltpu.VMEM((_CHUNK, _C), jnp.float32),
            pltpu.VMEM((_CHUNK, _C), jnp.float32),
            pltpu.VMEM((_CHUNK, _ACCW), jnp.float32),
            pltpu.VMEM((_H, _CH), jnp.float32),
            pltpu.VMEM((_ZROWS, _ACCW), jnp.float32),
            pltpu.VMEM_SHARED((_N, _ACCW), jnp.float32),
        ],
    )
    def k(xl_hbm, xr_hbm, xe_hbm, src_hbm, dst_hbm, att_hbm, out_hbm,
          sidx, didx, xlv, xrv, xev, ctr, attv, zv, acc):
        cid = lax.axis_index("c")
        sid = lax.axis_index("s")
        wid = sid * _NC + cid

        @pl.loop(0, _ZROWS)
        def _(r):
            for j in range(_ACCW // _CH):
                zv[r, pl.ds(j * _CH, _CH)] = jnp.zeros((_CH,), jnp.float32)

        for t in range(_RPS // _ZROWS):
            pltpu.sync_copy(zv, acc.at[pl.ds(sid * _RPS + t * _ZROWS, _ZROWS)])
        pltpu.sync_copy(att_hbm, attv)
        plsc.subcore_barrier()

        lanes = lax.iota(jnp.int32, _CH)

        @pl.loop(0, _NCHUNK)
        def _(i):
            base = wid * _EPW + i * _CHUNK
            pltpu.sync_copy(src_hbm.at[pl.ds(base, _CHUNK)], sidx)
            pltpu.sync_copy(dst_hbm.at[pl.ds(base, _CHUNK)], didx)
            pltpu.sync_copy(xl_hbm.at[sidx], xlv)
            pltpu.sync_copy(xr_hbm.at[didx], xrv)
            pltpu.sync_copy(xe_hbm.at[pl.ds(base, _CHUNK)], xev)

            @pl.loop(0, _CHUNK)
            def _(e):
                den = jnp.zeros((_CH,), jnp.float32)
                for h in range(_H):
                    sl = pl.ds(h * _CH, _CH)
                    ml = xlv[e, sl]
                    z = ml + xrv[e, sl] + xev[e, sl]
                    m = jnp.where(z >= 0.0, z, z * 0.2)
                    a = jnp.sum(m * attv[h, :])
                    ex = jnp.exp(jnp.full((_CH,), a, jnp.float32))
                    ctr[e, sl] = ex * ml
                    den = jnp.where(lanes == h, ex, den)
                ctr[e, pl.ds(_C, _CH)] = den

            pltpu.sync_copy(ctr, acc.at[didx], add=True)

        plsc.subcore_barrier()
        pltpu.sync_copy(
            acc.at[pl.ds(sid * _RPS, _RPS)],
            out_hbm.at[pl.ds(cid * _N + sid * _RPS, _RPS)],
        )

    return k(xl, xr, xe, src, dst, att)


def _combine(acc, bo, g, be, residual):
    def body(*refs):
        if residual is None:
            acc_ref, bo_ref, g_ref, be_ref, out_ref = refs
        else:
            acc_ref, bo_ref, g_ref, be_ref, res_ref, out_ref = refs
        num = acc_ref[pl.ds(0, _N), pl.ds(0, _C)] + acc_ref[pl.ds(_N, _N), pl.ds(0, _C)]
        den16 = (
            acc_ref[pl.ds(0, _N), pl.ds(_C, _CH)]
            + acc_ref[pl.ds(_N, _N), pl.ds(_C, _CH)]
        )
        row = lax.broadcasted_iota(jnp.int32, (_CH, _C), 0)
        col = lax.broadcasted_iota(jnp.int32, (_CH, _C), 1)
        sel = (row == col // _CH).astype(jnp.float32)
        den = jnp.dot(den16, sel, preferred_element_type=jnp.float32)
        outv = num / (den + 1e-16) + bo_ref[...]
        mu = jnp.mean(outv, axis=0, keepdims=True)
        var = jnp.mean((outv - mu) * (outv - mu), axis=0, keepdims=True)
        y = (outv - mu) / jnp.sqrt(var + 1e-5) * g_ref[...] + be_ref[...]
        if residual is not None:
            y = y + res_ref[...]
        out_ref[...] = jnp.maximum(y, 0.0)

    args = [acc, bo.reshape(1, _C), g.reshape(1, _C), be.reshape(1, _C)]
    if residual is not None:
        args.append(residual)
    return pl.pallas_call(
        body,
        out_shape=jax.ShapeDtypeStruct((_N, _C), jnp.float32),
    )(*args)


def _layer(x, xe, src, dst, Wl, bl, Wr, br, att, bo, g, be, residual):
    xl, xr = _node_transform(x, Wl, bl, Wr, br)
    acc = _sc_edge_pass(xl, xr, xe, src, dst, att)
    return _combine(acc, bo, g, be, residual)


def kernel(x, edge_index, edge_attr,
           Wl1, bl1, Wr1, br1, We1, att1, bo1, g1, be1,
           Wl2, bl2, Wr2, br2, We2, att2, bo2, g2, be2):
    src = edge_index[0].astype(jnp.int32)
    dst = edge_index[1].astype(jnp.int32)
    xe1 = _edge_transform(edge_attr, We1)
    xe2 = _edge_transform(edge_attr, We2)
    h = _layer(x, xe1, src, dst, Wl1, bl1, Wr1, br1, att1, bo1, g1, be1, None)
    out = _layer(h, xe2, src, dst, Wl2, bl2, Wr2, br2, att2, bo2, g2, be2, x)
    return out


# speedup vs baseline: 7.6996x; 7.6996x over previous
import dataclasses
import functools

import jax
import jax.numpy as jnp
from jax import lax
from jax.experimental import pallas as pl
from jax.experimental.pallas import tpu as pltpu
from jax.experimental.pallas import tpu_sc as plsc

_N = 10000
_E = 320000
_C = 128
_H = 8
_CH = 16
_ED = 16

_NC = 2
_NS = 16
_EPS = _E // _NS
_CHUNK = 80
_NCHUNK = _EPS // _CHUNK
_HALF = 5120
_CPR = 5128
_DUMP = _HALF
_RPZ = _HALF // _NS


def _node_transform(x, Wl, bl, Wr, br):
    def body(x_ref, wl_ref, bl_ref, wr_ref, br_ref, xl_ref, xr_ref):
        xv = x_ref[...]
        xl_ref[...] = (
            jnp.dot(xv, wl_ref[...], preferred_element_type=jnp.float32)
            + bl_ref[...]
        )
        xr_ref[...] = (
            jnp.dot(xv, wr_ref[...], preferred_element_type=jnp.float32)
            + br_ref[...]
        )

    return pl.pallas_call(
        body,
        out_shape=(
            jax.ShapeDtypeStruct((_N, _C), jnp.float32),
            jax.ShapeDtypeStruct((_N, _C), jnp.float32),
        ),
    )(x, Wl, bl.reshape(1, _C), Wr, br.reshape(1, _C))


def _edge_transform(ea, We):
    blk = 8000

    def body(ea_ref, we_ref, xe_ref):
        xe_ref[...] = jnp.dot(
            ea_ref[...], we_ref[...], preferred_element_type=jnp.float32
        )

    return pl.pallas_call(
        body,
        grid=(_E // blk,),
        in_specs=[
            pl.BlockSpec((blk, _ED), lambda i: (i, 0)),
            pl.BlockSpec((_ED, _C), lambda i: (0, 0)),
        ],
        out_specs=pl.BlockSpec((blk, _C), lambda i: (i, 0)),
        out_shape=jax.ShapeDtypeStruct((_E, _C), jnp.float32),
    )(ea, We)


def _build_sc_edge_pass():
    mesh = plsc.VectorSubcoreMesh(core_axis_name="c", subcore_axis_name="s")
    cp = pltpu.CompilerParams()
    if "needs_layout_passes" in pltpu.CompilerParams.__dataclass_fields__:
        cp = dataclasses.replace(cp, needs_layout_passes=False)

    @functools.partial(
        pl.kernel,
        out_type=(
            jax.ShapeDtypeStruct((_NC * _CPR, _C), jnp.float32),
            jax.ShapeDtypeStruct((_NC * _CPR, _C), jnp.float32),
            jax.ShapeDtypeStruct((_E, _CH), jnp.float32),
        ),
        mesh=mesh,
        compiler_params=cp,
        scratch_types=[
            pltpu.VMEM((_CHUNK,), jnp.int32),
            pltpu.VMEM((_CHUNK,), jnp.int32),
            pltpu.VMEM((_CHUNK,), jnp.int32),
            pltpu.VMEM((_CHUNK, _C), jnp.float32),
            pltpu.VMEM((_CHUNK, _C), jnp.float32),
            pltpu.VMEM((_CHUNK, _C), jnp.float32),
            pltpu.VMEM((_CHUNK, _C), jnp.float32),
            pltpu.VMEM((_CHUNK, _CH), jnp.float32),
            pltpu.VMEM((_H, _CH), jnp.float32),
            pltpu.VMEM((128, _C), jnp.float32),
            pltpu.VMEM_SHARED((_CPR, _C), jnp.float32),
        ],
    )
    def k(xl_hbm, xr_hbm, xe_hbm, src_hbm, dst_hbm, att_hbm,
          num_hbm, den_hbm, ex_hbm,
          sidx, didx, lidx, xlv, xrv, xev, ctr, exv, attv, zv, acc):
        cid = lax.axis_index("c")
        sid = lax.axis_index("s")
        row0 = sid * _RPZ
        base_node = cid * _HALF

        @pl.loop(0, 128)
        def _(r):
            for j in range(_C // _CH):
                zv[r, pl.ds(j * _CH, _CH)] = jnp.zeros((_CH,), jnp.float32)

        def zero_acc():
            for off, ln in ((0, 128), (128, 128), (256, 64)):
                pltpu.sync_copy(zv.at[pl.ds(0, ln)],
                                acc.at[pl.ds(row0 + off, ln)])

        def make_lidx():
            for j in range(_CHUNK // _CH):
                sl = pl.ds(j * _CH, _CH)
                lv = didx[sl] - base_node
                ok = (lv >= 0) & (lv < _HALF)
                lidx[sl] = jnp.where(ok, lv, _DUMP)

        zero_acc()
        pltpu.sync_copy(att_hbm, attv)
        plsc.subcore_barrier()

        @pl.loop(0, _NCHUNK)
        def _(i):
            ebase = sid * _EPS + i * _CHUNK
            pltpu.sync_copy(src_hbm.at[pl.ds(ebase, _CHUNK)], sidx)
            pltpu.sync_copy(dst_hbm.at[pl.ds(ebase, _CHUNK)], didx)
            pltpu.sync_copy(xl_hbm.at[sidx], xlv)
            pltpu.sync_copy(xr_hbm.at[didx], xrv)
            pltpu.sync_copy(xe_hbm.at[pl.ds(ebase, _CHUNK)], xev)
            make_lidx()

            @pl.loop(0, _CHUNK)
            def _(e):
                den = jnp.zeros((_CH,), jnp.float32)
                lanes = lax.iota(jnp.int32, _CH)
                for h in range(_H):
                    sl = pl.ds(h * _CH, _CH)
                    ml = xlv[e, sl]
                    z = ml + xrv[e, sl] + xev[e, sl]
                    m = jnp.where(z >= 0.0, z, z * 0.2)
                    a = jnp.sum(m * attv[h, :])
                    ex = jnp.exp(jnp.full((_CH,), a, jnp.float32))
                    ctr[e, sl] = ex * ml
                    den = jnp.where(lanes == h, ex, den)
                exv[e, :] = den

            pltpu.sync_copy(ctr, acc.at[lidx], add=True)
            pltpu.sync_copy(exv, ex_hbm.at[pl.ds(ebase, _CHUNK)])

        plsc.subcore_barrier()
        pltpu.sync_copy(
            acc.at[pl.ds(row0, _RPZ)],
            num_hbm.at[pl.ds(cid * _CPR + row0, _RPZ)],
        )
        zero_acc()
        plsc.subcore_barrier()

        @pl.loop(0, _CHUNK)
        def _(e):
            for j in range(1, _C // _CH):
                ctr[e, pl.ds(j * _CH, _CH)] = jnp.zeros((_CH,), jnp.float32)

        @pl.loop(0, _NCHUNK)
        def _(i):
            ebase = sid * _EPS + i * _CHUNK
            pltpu.sync_copy(dst_hbm.at[pl.ds(ebase, _CHUNK)], didx)
            pltpu.sync_copy(ex_hbm.at[pl.ds(ebase, _CHUNK)], exv)
            make_lidx()

            @pl.loop(0, _CHUNK)
            def _(e):
                ctr[e, pl.ds(0, _CH)] = exv[e, :]

            pltpu.sync_copy(ctr, acc.at[lidx], add=True)

        plsc.subcore_barrier()
        pltpu.sync_copy(
            acc.at[pl.ds(row0, _RPZ)],
            den_hbm.at[pl.ds(cid * _CPR + row0, _RPZ)],
        )

    return k


_sc_edge_pass = _build_sc_edge_pass()


def _combine(accn, accd, bo, g, be, residual):
    def body(*refs):
        if residual is None:
            num_ref, den_ref, bo_ref, g_ref, be_ref, out_ref = refs
        else:
            num_ref, den_ref, bo_ref, g_ref, be_ref, res_ref, out_ref = refs
        num = jnp.concatenate(
            [num_ref[pl.ds(0, _HALF), :],
             num_ref[pl.ds(_CPR, _N - _HALF), :]], axis=0)
        den16 = jnp.concatenate(
            [den_ref[pl.ds(0, _HALF), pl.ds(0, _CH)],
             den_ref[pl.ds(_CPR, _N - _HALF), pl.ds(0, _CH)]], axis=0)
        row = lax.broadcasted_iota(jnp.int32, (_CH, _C), 0)
        col = lax.broadcasted_iota(jnp.int32, (_CH, _C), 1)
        sel = (row == col // _CH).astype(jnp.float32)
        den = jnp.dot(den16, sel, preferred_element_type=jnp.float32)
        outv = num / (den + 1e-16) + bo_ref[...]
        mu = jnp.mean(outv, axis=0, keepdims=True)
        var = jnp.mean((outv - mu) * (outv - mu), axis=0, keepdims=True)
        y = (outv - mu) / jnp.sqrt(var + 1e-5) * g_ref[...] + be_ref[...]
        if residual is not None:
            y = y + res_ref[...]
        out_ref[...] = jnp.maximum(y, 0.0)

    args = [accn, accd, bo.reshape(1, _C), g.reshape(1, _C), be.reshape(1, _C)]
    if residual is not None:
        args.append(residual)
    return pl.pallas_call(
        body,
        out_shape=jax.ShapeDtypeStruct((_N, _C), jnp.float32),
    )(*args)


def _layer(x, xe, src, dst, Wl, bl, Wr, br, att, bo, g, be, residual):
    xl, xr = _node_transform(x, Wl, bl, Wr, br)
    accn, accd, _ = _sc_edge_pass(xl, xr, xe, src, dst, att)
    return _combine(accn, accd, bo, g, be, residual)


def kernel(x, edge_index, edge_attr,
           Wl1, bl1, Wr1, br1, We1, att1, bo1, g1, be1,
           Wl2, bl2, Wr2, br2, We2, att2, bo2, g2, be2):
    src = edge_index[0].astype(jnp.int32)
    dst = edge_index[1].astype(jnp.int32)
    xe1 = _edge_transform(edge_attr, We1)
    xe2 = _edge_transform(edge_attr, We2)
    h = _layer(x, xe1, src, dst, Wl1, bl1, Wr1, br1, att1, bo1, g1, be1, None)
    out = _layer(h, xe2, src, dst, Wl2, bl2, Wr2, br2, att2, bo2, g2, be2, x)
    return out


# speedup vs baseline: 30.7713x; 3.9965x over previous
import dataclasses
import functools

import jax
import jax.numpy as jnp
from jax import lax
from jax.experimental import pallas as pl
from jax.experimental.pallas import tpu as pltpu
from jax.experimental.pallas import tpu_sc as plsc

_N = 10000
_E = 320000
_C = 128
_H = 8
_CH = 16
_ED = 16

_NC = 2
_NS = 16
_EPS = _E // _NS
_CHUNK = 80
_NCHUNK = _EPS // _CHUNK
_HALF = 5120
_CPR = 5128
_DUMP = _HALF
_RPZ = _HALF // _NS


def _node_transform(x, Wl, bl, Wr, br):
    def body(x_ref, wl_ref, bl_ref, wr_ref, br_ref, xl_ref, xr_ref):
        xv = x_ref[...]
        xl_ref[...] = (
            jnp.dot(xv, wl_ref[...], preferred_element_type=jnp.float32)
            + bl_ref[...]
        )
        xr_ref[...] = (
            jnp.dot(xv, wr_ref[...], preferred_element_type=jnp.float32)
            + br_ref[...]
        )

    return pl.pallas_call(
        body,
        out_shape=(
            jax.ShapeDtypeStruct((_N, _C), jnp.float32),
            jax.ShapeDtypeStruct((_N, _C), jnp.float32),
        ),
    )(x, Wl, bl.reshape(1, _C), Wr, br.reshape(1, _C))


def _edge_transform(ea, We):
    blk = 8000

    def body(ea_ref, we_ref, xe_ref):
        xe_ref[...] = jnp.dot(
            ea_ref[...], we_ref[...], preferred_element_type=jnp.float32
        )

    return pl.pallas_call(
        body,
        grid=(_E // blk,),
        in_specs=[
            pl.BlockSpec((blk, _ED), lambda i: (i, 0)),
            pl.BlockSpec((_ED, _C), lambda i: (0, 0)),
        ],
        out_specs=pl.BlockSpec((blk, _C), lambda i: (i, 0)),
        out_shape=jax.ShapeDtypeStruct((_E, _C), jnp.float32),
    )(ea, We)


def _build_sc_edge_pass():
    mesh = plsc.VectorSubcoreMesh(core_axis_name="c", subcore_axis_name="s")
    cp = pltpu.CompilerParams()
    if "needs_layout_passes" in pltpu.CompilerParams.__dataclass_fields__:
        cp = dataclasses.replace(cp, needs_layout_passes=False)

    ivec = functools.partial(pltpu.VMEM, (_CHUNK,))
    rows = functools.partial(pltpu.VMEM, (_CHUNK, _C))
    exrow = functools.partial(pltpu.VMEM, (_CHUNK * _CH,))

    @functools.partial(
        pl.kernel,
        out_type=(
            jax.ShapeDtypeStruct((_NC * _CPR, _C), jnp.float32),
            jax.ShapeDtypeStruct((_NC * _CPR, _C), jnp.float32),
            jax.ShapeDtypeStruct((_E * _CH,), jnp.float32),
        ),
        mesh=mesh,
        compiler_params=cp,
        scratch_types=[
            ivec(jnp.int32), ivec(jnp.int32),
            ivec(jnp.int32), ivec(jnp.int32),
            ivec(jnp.int32), ivec(jnp.int32),
            rows(jnp.float32), rows(jnp.float32),
            rows(jnp.float32), rows(jnp.float32),
            rows(jnp.float32), rows(jnp.float32),
            exrow(jnp.float32), exrow(jnp.float32),
            pltpu.VMEM((_H, _CH), jnp.float32),
            pltpu.VMEM((32, _C), jnp.float32),
            pltpu.VMEM_SHARED((_CPR, _C), jnp.float32),
        ] + [pltpu.SemaphoreType.DMA] * 10,
    )
    def k(xl_hbm, xr_hbm, xe_hbm, src_hbm, dst_hbm, att_hbm,
          num_hbm, den_hbm, ex_hbm,
          sidx0, sidx1, didx0, didx1, lidx0, lidx1,
          xlv0, xlv1, xrv0, xrv1, ctr0, ctr1, exv0, exv1,
          attv, zv, acc,
          si0, si1, sg0, sg1, ss0, ss1, sx0, sx1, se0, se1):
        SIDX = (sidx0, sidx1)
        DIDX = (didx0, didx1)
        LIDX = (lidx0, lidx1)
        XLV = (xlv0, xlv1)
        XRV = (xrv0, xrv1)
        CTR = (ctr0, ctr1)
        EXV = (exv0, exv1)
        SI = (si0, si1)
        SG = (sg0, sg1)
        SS = (ss0, ss1)
        SX = (sx0, sx1)
        SE = (se0, se1)

        cid = lax.axis_index("c")
        sid = lax.axis_index("s")
        row0 = sid * _RPZ
        base_node = cid * _HALF

        @pl.loop(0, 32)
        def _(r):
            for j in range(_C // _CH):
                zv[r, pl.ds(j * _CH, _CH)] = jnp.zeros((_CH,), jnp.float32)

        def zero_acc():
            @pl.loop(0, _RPZ // 32)
            def _(t):
                pltpu.sync_copy(zv, acc.at[pl.ds(row0 + t * 32, 32)])

        def ebase(i):
            return sid * _EPS + i * _CHUNK

        def make_lidx(p):
            for j in range(_CHUNK // _CH):
                sl = pl.ds(j * _CH, _CH)
                lv = DIDX[p][sl] - base_node
                ok = (lv >= 0) & (lv < _HALF)
                LIDX[p][sl] = jnp.where(ok, lv, _DUMP)

        def start_idx(i, p, with_src):
            if with_src:
                pltpu.async_copy(src_hbm.at[pl.ds(ebase(i), _CHUNK)],
                                 SIDX[p], SI[p])
            pltpu.async_copy(dst_hbm.at[pl.ds(ebase(i), _CHUNK)],
                             DIDX[p], SI[p])

        def wait_idx(i, p, with_src):
            if with_src:
                pltpu.make_async_copy(src_hbm.at[pl.ds(ebase(i), _CHUNK)],
                                      SIDX[p], SI[p]).wait()
            pltpu.make_async_copy(dst_hbm.at[pl.ds(ebase(i), _CHUNK)],
                                  DIDX[p], SI[p]).wait()

        pltpu.sync_copy(att_hbm, attv)
        zero_acc()
        plsc.subcore_barrier()
        atts = [attv[h, :] for h in range(_H)]
        lanes = lax.iota(jnp.int32, _CH)

        def p1_fire_g(i, p):
            hxl = pltpu.async_copy(xl_hbm.at[SIDX[p]], XLV[p], SG[p])
            hxr = pltpu.async_copy(xr_hbm.at[DIDX[p]], XRV[p], SG[p])
            return hxl, hxr

        def p1_fire_xe(i, p):
            pltpu.async_copy(xe_hbm.at[pl.ds(ebase(i), _CHUNK)], CTR[p], SX[p])

        def p1_wait_xe(i, p):
            pltpu.make_async_copy(xe_hbm.at[pl.ds(ebase(i), _CHUNK)],
                                  CTR[p], SX[p]).wait()

        def p1_fire_ex(i, p):
            pltpu.async_copy(EXV[p],
                             ex_hbm.at[pl.ds(ebase(i) * _CH, _CHUNK * _CH)],
                             SE[p])

        def p1_wait_ex(i, p):
            pltpu.make_async_copy(EXV[p],
                                  ex_hbm.at[pl.ds(ebase(i) * _CH,
                                                  _CHUNK * _CH)],
                                  SE[p]).wait()

        def p1_compute(p):
            @pl.loop(0, _CHUNK)
            def _(e):
                den = jnp.zeros((_CH,), jnp.float32)
                for h in range(_H):
                    sl = pl.ds(h * _CH, _CH)
                    ml = XLV[p][e, sl]
                    z = ml + XRV[p][e, sl] + CTR[p][e, sl]
                    m = jnp.where(z >= 0.0, z, z * 0.2)
                    a = jnp.sum(m * atts[h])
                    ex = jnp.exp(jnp.full((_CH,), a, jnp.float32))
                    CTR[p][e, sl] = ex * ml
                    den = jnp.where(lanes == h, ex, den)
                EXV[p][pl.ds(e * _CH, _CH)] = den

        start_idx(0, 0, True)
        wait_idx(0, 0, True)
        hg = p1_fire_g(0, 0)
        p1_fire_xe(0, 0)
        start_idx(1, 1, True)
        hg[0].wait()
        hg[1].wait()

        @pl.loop(0, _NCHUNK // 2)
        def _(j):
            for p in (0, 1):
                i = 2 * j + p
                q = 1 - p

                @pl.when(i >= 1)
                def _():
                    p1_wait_ex(i - 1, q)

                @pl.when(i + 1 < _NCHUNK)
                def _():
                    wait_idx(i + 1, q, True)

                hg = [None]

                @pl.when(i + 1 < _NCHUNK)
                def _():
                    hg[0] = p1_fire_g(i + 1, q)
                    p1_fire_xe(i + 1, q)

                p1_wait_xe(i, p)
                p1_compute(p)
                make_lidx(p)
                hs = pltpu.async_copy(CTR[p], acc.at[LIDX[p]], SS[p],
                                      add=True)
                p1_fire_ex(i, p)

                @pl.when(i + 2 < _NCHUNK)
                def _():
                    start_idx(i + 2, p, True)

                @pl.when(i + 1 < _NCHUNK)
                def _():
                    hg[0][0].wait()
                    hg[0][1].wait()

                hs.wait()

        p1_wait_ex(_NCHUNK - 1, 1)

        plsc.subcore_barrier()
        pltpu.sync_copy(
            acc.at[pl.ds(row0, _RPZ)],
            num_hbm.at[pl.ds(cid * _CPR + row0, _RPZ)],
        )
        zero_acc()
        plsc.subcore_barrier()

        @pl.loop(0, _CHUNK)
        def _(e):
            for p in (0, 1):
                for j in range(1, _C // _CH):
                    CTR[p][e, pl.ds(j * _CH, _CH)] = (
                        jnp.zeros((_CH,), jnp.float32))

        def p2_start_in(i, p):
            start_idx(i, p, False)
            pltpu.async_copy(ex_hbm.at[pl.ds(ebase(i) * _CH, _CHUNK * _CH)],
                             EXV[p], SX[p])

        def p2_wait_in(i, p):
            wait_idx(i, p, False)
            pltpu.make_async_copy(ex_hbm.at[pl.ds(ebase(i) * _CH,
                                                  _CHUNK * _CH)],
                                  EXV[p], SX[p]).wait()

        def p2_prep(i, p):
            p2_wait_in(i, p)
            make_lidx(p)

            @pl.loop(0, _CHUNK)
            def _(e):
                CTR[p][e, pl.ds(0, _CH)] = EXV[p][pl.ds(e * _CH, _CH)]

            return pltpu.async_copy(CTR[p], acc.at[LIDX[p]], SS[p], add=True)

        p2_start_in(0, 0)
        p2_start_in(1, 1)

        @pl.loop(0, _NCHUNK // 2)
        def _(j):
            h0 = p2_prep(2 * j, 0)
            h1 = p2_prep(2 * j + 1, 1)

            @pl.when(2 * j + 2 < _NCHUNK)
            def _():
                p2_start_in(2 * j + 2, 0)

            @pl.when(2 * j + 3 < _NCHUNK)
            def _():
                p2_start_in(2 * j + 3, 1)

            h0.wait()
            h1.wait()

        plsc.subcore_barrier()
        pltpu.sync_copy(
            acc.at[pl.ds(row0, _RPZ)],
            den_hbm.at[pl.ds(cid * _CPR + row0, _RPZ)],
        )

    return k


_sc_edge_pass = _build_sc_edge_pass()


def _combine(accn, accd, bo, g, be, residual):
    def body(*refs):
        if residual is None:
            num_ref, den_ref, bo_ref, g_ref, be_ref, out_ref = refs
        else:
            num_ref, den_ref, bo_ref, g_ref, be_ref, res_ref, out_ref = refs
        num = jnp.concatenate(
            [num_ref[pl.ds(0, _HALF), :],
             num_ref[pl.ds(_CPR, _N - _HALF), :]], axis=0)
        den16 = jnp.concatenate(
            [den_ref[pl.ds(0, _HALF), pl.ds(0, _CH)],
             den_ref[pl.ds(_CPR, _N - _HALF), pl.ds(0, _CH)]], axis=0)
        row = lax.broadcasted_iota(jnp.int32, (_CH, _C), 0)
        col = lax.broadcasted_iota(jnp.int32, (_CH, _C), 1)
        sel = (row == col // _CH).astype(jnp.float32)
        den = jnp.dot(den16, sel, preferred_element_type=jnp.float32)
        outv = num / (den + 1e-16) + bo_ref[...]
        mu = jnp.mean(outv, axis=0, keepdims=True)
        var = jnp.mean((outv - mu) * (outv - mu), axis=0, keepdims=True)
        y = (outv - mu) / jnp.sqrt(var + 1e-5) * g_ref[...] + be_ref[...]
        if residual is not None:
            y = y + res_ref[...]
        out_ref[...] = jnp.maximum(y, 0.0)

    args = [accn, accd, bo.reshape(1, _C), g.reshape(1, _C), be.reshape(1, _C)]
    if residual is not None:
        args.append(residual)
    return pl.pallas_call(
        body,
        out_shape=jax.ShapeDtypeStruct((_N, _C), jnp.float32),
    )(*args)


def _layer(x, xe, src, dst, Wl, bl, Wr, br, att, bo, g, be, residual):
    xl, xr = _node_transform(x, Wl, bl, Wr, br)
    accn, accd, _ = _sc_edge_pass(xl, xr, xe, src, dst, att)
    return _combine(accn, accd, bo, g, be, residual)


def kernel(x, edge_index, edge_attr,
           Wl1, bl1, Wr1, br1, We1, att1, bo1, g1, be1,
           Wl2, bl2, Wr2, br2, We2, att2, bo2, g2, be2):
    src = edge_index[0].astype(jnp.int32)
    dst = edge_index[1].astype(jnp.int32)
    xe1 = _edge_transform(edge_attr, We1)
    xe2 = _edge_transform(edge_attr, We2)
    h = _layer(x, xe1, src, dst, Wl1, bl1, Wr1, br1, att1, bo1, g1, be1, None)
    out = _layer(h, xe2, src, dst, Wl2, bl2, Wr2, br2, att2, bo2, g2, be2, x)
    return out
